# Initial kernel scaffold; baseline (speedup 1.0000x reference)
import jax
import jax.numpy as jnp
from jax.experimental import pallas as pl


def kernel(cards, emb):
    raise NotImplementedError("write your pallas kernel here")


# speedup vs baseline: 1.4658x; 1.4658x over previous
import functools

import jax
import jax.numpy as jnp
from jax import lax
from jax.experimental import pallas as pl
from jax.experimental.pallas import tpu as pltpu
from jax.experimental.pallas import tpu_sc as plsc

NUM_CARDS = 1000000
HIDDEN_DIM = 32
BATCH = 4096
HIST = 200

_info = plsc.get_sparse_core_info()
NC, NS, L = _info.num_cores, _info.num_subcores, _info.num_lanes
NW = NC * NS

B = BATCH * HIST
B_PER_W = B // NW
CHUNK = 1600
NCHUNK = B_PER_W // CHUNK


def _body(cards_hbm, emb_hbm, out_hbm, idx_v, rows_v, sem):
    wid = lax.axis_index("s") * NC + lax.axis_index("c")
    base = wid * B_PER_W

    def chunk_step(c, _):
        off = base + c * CHUNK
        pltpu.sync_copy(cards_hbm.at[pl.ds(off, CHUNK)], idx_v)

        def add1(i, _):
            sl = pl.ds(i * L, L)
            idx_v[sl] = idx_v[sl] + 1
            return 0

        lax.fori_loop(0, CHUNK // L, add1, 0)

        pltpu.async_copy(emb_hbm.at[idx_v], rows_v, sem).wait()
        pltpu.sync_copy(rows_v, out_hbm.at[pl.ds(off, CHUNK)])
        return 0

    lax.fori_loop(0, NCHUNK, chunk_step, 0)


@jax.jit
def _embed(cards_flat, emb):
    mesh = plsc.VectorSubcoreMesh(core_axis_name="c", subcore_axis_name="s")
    fn = pl.kernel(
        _body,
        out_type=jax.ShapeDtypeStruct((B, HIDDEN_DIM), jnp.float32),
        mesh=mesh,
        scratch_types=[
            pltpu.VMEM((CHUNK,), jnp.int32),
            pltpu.VMEM((CHUNK, HIDDEN_DIM), jnp.float32),
            pltpu.SemaphoreType.DMA,
        ],
        compiler_params=pltpu.CompilerParams(use_tc_tiling_on_sc=False),
    )
    return fn(cards_flat, emb)


def kernel(cards, emb):
    cards_flat = cards.reshape(-1).astype(jnp.int32)
    out = _embed(cards_flat, emb)
    return out.reshape(BATCH, HIST, HIDDEN_DIM)


# speedup vs baseline: 1.5002x; 1.0234x over previous
import functools

import jax
import jax.numpy as jnp
from jax import lax
from jax.experimental import pallas as pl
from jax.experimental.pallas import tpu as pltpu
from jax.experimental.pallas import tpu_sc as plsc

NUM_CARDS = 1000000
HIDDEN_DIM = 32
BATCH = 4096
HIST = 200

_info = plsc.get_sparse_core_info()
NC, NS, L = _info.num_cores, _info.num_subcores, _info.num_lanes
NW = NC * NS

B = BATCH * HIST
B_PER_W = B // NW
CHUNK = 1600
NCHUNK = B_PER_W // CHUNK
NPAIR = NCHUNK // 2


def _body(cards_hbm, emb_hbm, out_hbm, idx_all, rows0, rows1, sg0, sg1, so0, so1):
    wid = lax.axis_index("s") * NC + lax.axis_index("c")
    base = wid * B_PER_W
    pltpu.sync_copy(cards_hbm.at[pl.ds(base, B_PER_W)], idx_all)

    def add1_chunk(c):
        def add1(i, _):
            sl = pl.ds(c * CHUNK + i * L, L)
            idx_all[sl] = idx_all[sl] + 1
            return 0

        lax.fori_loop(0, CHUNK // L, add1, 0, unroll=4)

    def idx_slice(c):
        return idx_all.at[pl.ds(pl.multiple_of(c * CHUNK, CHUNK), CHUNK)]

    def out_slice(c):
        return out_hbm.at[pl.ds(pl.multiple_of(base + c * CHUNK, CHUNK), CHUNK)]

    def start_gather(c, rows, sem):
        pltpu.async_copy(emb_hbm.at[idx_slice(c)], rows, sem)

    def wait_gather(c, rows, sem):
        pltpu.make_async_copy(emb_hbm.at[idx_slice(c)], rows, sem).wait()

    def start_write(c, rows, sem):
        pltpu.async_copy(rows, out_slice(c), sem)

    def wait_write(c, rows, sem):
        pltpu.make_async_copy(rows, out_slice(c), sem).wait()

    add1_chunk(0)
    start_gather(0, rows0, sg0)
    add1_chunk(1)

    def pair(g, _):
        c0 = 2 * g
        c1 = c0 + 1

        @pl.when(g >= 1)
        def _():
            wait_write(c0 - 1, rows1, so1)

        start_gather(c1, rows1, sg1)

        @pl.when(g < NPAIR - 1)
        def _():
            add1_chunk(c0 + 2)

        wait_gather(c0, rows0, sg0)
        start_write(c0, rows0, so0)

        @pl.when(g < NPAIR - 1)
        def _():
            wait_write(c0, rows0, so0)
            start_gather(c1 + 1, rows0, sg0)
            add1_chunk(c1 + 2)

        wait_gather(c1, rows1, sg1)
        start_write(c1, rows1, so1)
        return 0

    lax.fori_loop(0, NPAIR, pair, 0)

    wait_write(NCHUNK - 2, rows0, so0)
    wait_write(NCHUNK - 1, rows1, so1)


@jax.jit
def _embed(cards_flat, emb):
    mesh = plsc.VectorSubcoreMesh(core_axis_name="c", subcore_axis_name="s")
    fn = pl.kernel(
        _body,
        out_type=jax.ShapeDtypeStruct((B, HIDDEN_DIM), jnp.float32),
        mesh=mesh,
        scratch_types=[
            pltpu.VMEM((B_PER_W,), jnp.int32),
            pltpu.VMEM((CHUNK, HIDDEN_DIM), jnp.float32),
            pltpu.VMEM((CHUNK, HIDDEN_DIM), jnp.float32),
            pltpu.SemaphoreType.DMA,
            pltpu.SemaphoreType.DMA,
            pltpu.SemaphoreType.DMA,
            pltpu.SemaphoreType.DMA,
        ],
        compiler_params=pltpu.CompilerParams(use_tc_tiling_on_sc=False),
    )
    return fn(cards_flat, emb)


def kernel(cards, emb):
    cards_flat = cards.reshape(-1).astype(jnp.int32)
    out = _embed(cards_flat, emb)
    return out.reshape(BATCH, HIST, HIDDEN_DIM)


# speedup vs baseline: 1.5015x; 1.0009x over previous
import functools

import jax
import jax.numpy as jnp
from jax import lax
from jax.experimental import pallas as pl
from jax.experimental.pallas import tpu as pltpu
from jax.experimental.pallas import tpu_sc as plsc

NUM_CARDS = 1000000
HIDDEN_DIM = 32
BATCH = 4096
HIST = 200

_info = plsc.get_sparse_core_info()
NC, NS, L = _info.num_cores, _info.num_subcores, _info.num_lanes
NW = NC * NS

B = BATCH * HIST
B_PER_W = B // NW
CHUNK = 1600
NCHUNK = B_PER_W // CHUNK
NPAIR = NCHUNK // 2


def _body(cards_hbm, emb_hbm, out_hbm, idx_all, rows0, rows1, sg0, sg1, so0, so1):
    wid = lax.axis_index("s") * NC + lax.axis_index("c")
    base = wid * B_PER_W
    pltpu.sync_copy(cards_hbm.at[pl.ds(base, B_PER_W)], idx_all)

    def add1_chunk(c):
        def add1(i, _):
            sl = pl.ds(c * CHUNK + i * L, L)
            idx_all[sl] = idx_all[sl] + 1
            return 0

        lax.fori_loop(0, CHUNK // L, add1, 0, unroll=4)

    def idx_slice(c):
        return idx_all.at[pl.ds(pl.multiple_of(c * CHUNK, CHUNK), CHUNK)]

    def start_gather(c, rows, sem):
        pltpu.async_copy(emb_hbm.at[idx_slice(c)], rows, sem)

    def wait_gather(c, rows, sem):
        pltpu.make_async_copy(emb_hbm.at[idx_slice(c)], rows, sem).wait()

    def start_write(c, rows, sem):
        b0 = (base + c * CHUNK) // HIST
        for k in range(CHUNK // HIST):
            pltpu.async_copy(
                rows.at[pl.ds(k * HIST, HIST)], out_hbm.at[b0 + k], sem
            )

    def wait_write(c, rows, sem):
        b0 = (base + c * CHUNK) // HIST
        for k in range(CHUNK // HIST):
            pltpu.make_async_copy(
                rows.at[pl.ds(k * HIST, HIST)], out_hbm.at[b0 + k], sem
            ).wait()

    add1_chunk(0)
    start_gather(0, rows0, sg0)
    add1_chunk(1)

    def pair(g, _):
        c0 = 2 * g
        c1 = c0 + 1

        @pl.when(g >= 1)
        def _():
            wait_write(c0 - 1, rows1, so1)

        start_gather(c1, rows1, sg1)

        @pl.when(g < NPAIR - 1)
        def _():
            add1_chunk(c0 + 2)

        wait_gather(c0, rows0, sg0)
        start_write(c0, rows0, so0)

        @pl.when(g < NPAIR - 1)
        def _():
            wait_write(c0, rows0, so0)
            start_gather(c1 + 1, rows0, sg0)
            add1_chunk(c1 + 2)

        wait_gather(c1, rows1, sg1)
        start_write(c1, rows1, so1)
        return 0

    lax.fori_loop(0, NPAIR, pair, 0)

    wait_write(NCHUNK - 2, rows0, so0)
    wait_write(NCHUNK - 1, rows1, so1)


@jax.jit
def _embed(cards_flat, emb):
    mesh = plsc.VectorSubcoreMesh(core_axis_name="c", subcore_axis_name="s")
    fn = pl.kernel(
        _body,
        out_type=jax.ShapeDtypeStruct((BATCH, HIST, HIDDEN_DIM), jnp.float32),
        mesh=mesh,
        scratch_types=[
            pltpu.VMEM((B_PER_W,), jnp.int32),
            pltpu.VMEM((CHUNK, HIDDEN_DIM), jnp.float32),
            pltpu.VMEM((CHUNK, HIDDEN_DIM), jnp.float32),
            pltpu.SemaphoreType.DMA,
            pltpu.SemaphoreType.DMA,
            pltpu.SemaphoreType.DMA,
            pltpu.SemaphoreType.DMA,
        ],
        compiler_params=pltpu.CompilerParams(use_tc_tiling_on_sc=False),
    )
    return fn(cards_flat, emb)


def kernel(cards, emb):
    cards_flat = cards.reshape(-1).astype(jnp.int32)
    return _embed(cards_flat, emb)
